# baseline (speedup 1.0000x reference)
import functools

import jax
import jax.numpy as jnp
from jax import lax
from jax.experimental import pallas as pl
from jax.experimental.pallas import tpu as pltpu
from jax.experimental.pallas import tpu_sc as plsc

B = 16384
D = 128
NC = 2
NS = 16
NW = NC * NS
BPW = B // NW
CHS = (16, 48, 64, 96, 96, 96, 96)
OFFS = (0, 16, 64, 128, 224, 320, 416)
NCHUNK = len(CHS)
CHMAX = max(CHS)
NBUF = 4
LANES = 16

_mesh = plsc.VectorSubcoreMesh(core_axis_name="c", subcore_axis_name="s")


@functools.partial(
    pl.kernel,
    mesh=_mesh,
    out_type=jax.ShapeDtypeStruct((B,), jnp.float32),
    scratch_types=[
        pltpu.VMEM((BPW,), jnp.int32),
        pltpu.VMEM((BPW,), jnp.int32),
        pltpu.VMEM((NBUF, CHMAX, D), jnp.float32),
        pltpu.VMEM((NBUF, CHMAX, D), jnp.float32),
        pltpu.VMEM((BPW,), jnp.float32),
        pltpu.SemaphoreType.DMA,
        pltpu.SemaphoreType.DMA,
        pltpu.SemaphoreType.DMA,
        pltpu.SemaphoreType.DMA,
        pltpu.SemaphoreType.DMA,
        pltpu.SemaphoreType.DMA,
        pltpu.SemaphoreType.DMA,
    ],
)
def _mf_kernel(users_hbm, movies_hbm, uf_hbm, mf_hbm, out_hbm,
               uidx_v, midx_v, u_v, m_v, out_v,
               sem0, sem1, sem2, sem3, sem4, sem5, sem6):
    wid = lax.axis_index("s") * NC + lax.axis_index("c")
    base = wid * BPW
    C0 = CHS[0]
    s0 = pltpu.async_copy(users_hbm.at[pl.ds(base, C0)],
                          uidx_v.at[pl.ds(0, C0)], sem0)
    s1 = pltpu.async_copy(movies_hbm.at[pl.ds(base, C0)],
                          midx_v.at[pl.ds(0, C0)], sem0)
    REST = BPW - C0
    s2 = pltpu.async_copy(users_hbm.at[pl.ds(base + C0, REST)],
                          uidx_v.at[pl.ds(C0, REST)], sem1)
    s3 = pltpu.async_copy(movies_hbm.at[pl.ds(base + C0, REST)],
                          midx_v.at[pl.ds(C0, REST)], sem1)

    lane_ids = lax.iota(jnp.int32, LANES)

    sems = (sem2, sem3, sem4, sem5)

    def start_gather(c):
        bc = c % NBUF
        n = CHS[c]
        nsplit = 2 if n >= 96 else 1
        h = n // nsplit
        copies = []
        for i in range(nsplit):
            copies.append(pltpu.async_copy(
                uf_hbm.at[uidx_v.at[pl.ds(OFFS[c] + i * h, h)]],
                u_v.at[bc, pl.ds(i * h, h)], sems[bc]))
            copies.append(pltpu.async_copy(
                mf_hbm.at[midx_v.at[pl.ds(OFFS[c] + i * h, h)]],
                m_v.at[bc, pl.ds(i * h, h)], sems[bc]))
        return tuple(copies)

    masks = {d: (lane_ids & d) == 0 for d in (8, 4, 2, 1)}

    def comb(a, b, d):
        m = masks[d]
        return (jnp.where(m, a, b)
                + jnp.where(m, b, a)
                .at[lane_ids ^ d].get(mode="promise_in_bounds"))

    def compute_chunk(c):
        bc = c % NBUF

        @plsc.parallel_loop(0, CHS[c] // 8,
                            carry=jnp.zeros((LANES,), jnp.float32))
        def half_body(h, carry):
            r0 = h * 8

            def dot_acc(j):
                r = r0 + j
                acc = (u_v[bc, r, pl.ds(0, LANES)]
                       * m_v[bc, r, pl.ds(0, LANES)])
                for k in range(1, D // LANES):
                    acc = acc + (u_v[bc, r, pl.ds(k * LANES, LANES)]
                                 * m_v[bc, r, pl.ds(k * LANES, LANES)])
                return acc

            w = comb(comb(comb(dot_acc(0), dot_acc(4), 4),
                          comb(dot_acc(2), dot_acc(6), 4), 2),
                     comb(comb(dot_acc(1), dot_acc(5), 4),
                          comb(dot_acc(3), dot_acc(7), 4), 2), 1)

            @pl.when(h & 1 == 1)
            def _():
                out_v[pl.ds(OFFS[c] + r0 - 8, LANES)] = comb(carry, w, 8)

            return w

    descs = [None] * NCHUNK
    s0.wait()
    s1.wait()
    descs[0] = start_gather(0)
    s2.wait()
    s3.wait()
    for c in range(1, min(NBUF, NCHUNK)):
        descs[c] = start_gather(c)
    outs = []
    for c in range(NCHUNK):
        for dsc in descs[c]:
            dsc.wait()
        compute_chunk(c)
        if c + NBUF < NCHUNK:
            descs[c + NBUF] = start_gather(c + NBUF)
        outs.append(pltpu.async_copy(
            out_v.at[pl.ds(OFFS[c], CHS[c])],
            out_hbm.at[pl.ds(base + OFFS[c], CHS[c])], sem6))
    for oc in outs:
        oc.wait()


def kernel(data, user_factors, movie_factors):
    users = data[:, 0].astype(jnp.int32)
    movies = data[:, 1].astype(jnp.int32)
    return _mf_kernel(users, movies, user_factors, movie_factors)


# speedup vs baseline: 1.0234x; 1.0234x over previous
import functools

import jax
import jax.numpy as jnp
from jax import lax
from jax.experimental import pallas as pl
from jax.experimental.pallas import tpu as pltpu
from jax.experimental.pallas import tpu_sc as plsc

B = 16384
D = 128
NC = 2
NS = 16
NW = NC * NS
BPW = B // NW
CHS = (16, 48, 64, 128, 128, 128)
OFFS = (0, 16, 64, 128, 256, 384)
NCHUNK = len(CHS)
CHMAX = max(CHS)
NBUF = 3
LANES = 16

_mesh = plsc.VectorSubcoreMesh(core_axis_name="c", subcore_axis_name="s")


@functools.partial(
    pl.kernel,
    mesh=_mesh,
    out_type=jax.ShapeDtypeStruct((B,), jnp.float32),
    scratch_types=[
        pltpu.VMEM((BPW,), jnp.int32),
        pltpu.VMEM((BPW,), jnp.int32),
        pltpu.VMEM((NBUF, CHMAX, D), jnp.float32),
        pltpu.VMEM((NBUF, CHMAX, D), jnp.float32),
        pltpu.VMEM((BPW,), jnp.float32),
        pltpu.SemaphoreType.DMA,
        pltpu.SemaphoreType.DMA,
        pltpu.SemaphoreType.DMA,
        pltpu.SemaphoreType.DMA,
        pltpu.SemaphoreType.DMA,
        pltpu.SemaphoreType.DMA,
        pltpu.SemaphoreType.DMA,
    ],
)
def _mf_kernel(users_hbm, movies_hbm, uf_hbm, mf_hbm, out_hbm,
               uidx_v, midx_v, u_v, m_v, out_v,
               sem0, sem1, sem2, sem3, sem4, sem5, sem6):
    wid = lax.axis_index("s") * NC + lax.axis_index("c")
    base = wid * BPW
    C0 = CHS[0]
    s0 = pltpu.async_copy(users_hbm.at[pl.ds(base, C0)],
                          uidx_v.at[pl.ds(0, C0)], sem0)
    s1 = pltpu.async_copy(movies_hbm.at[pl.ds(base, C0)],
                          midx_v.at[pl.ds(0, C0)], sem0)
    REST = BPW - C0
    s2 = pltpu.async_copy(users_hbm.at[pl.ds(base + C0, REST)],
                          uidx_v.at[pl.ds(C0, REST)], sem1)
    s3 = pltpu.async_copy(movies_hbm.at[pl.ds(base + C0, REST)],
                          midx_v.at[pl.ds(C0, REST)], sem1)

    lane_ids = lax.iota(jnp.int32, LANES)

    sems = (sem2, sem3, sem4)

    def start_gather(c):
        bc = c % NBUF
        n = CHS[c]
        nsplit = 2 if n >= 96 else 1
        h = n // nsplit
        copies = []
        for i in range(nsplit):
            copies.append(pltpu.async_copy(
                uf_hbm.at[uidx_v.at[pl.ds(OFFS[c] + i * h, h)]],
                u_v.at[bc, pl.ds(i * h, h)], sems[bc]))
            copies.append(pltpu.async_copy(
                mf_hbm.at[midx_v.at[pl.ds(OFFS[c] + i * h, h)]],
                m_v.at[bc, pl.ds(i * h, h)], sems[bc]))
        return tuple(copies)

    masks = {d: (lane_ids & d) == 0 for d in (8, 4, 2, 1)}

    def comb(a, b, d):
        m = masks[d]
        return (jnp.where(m, a, b)
                + jnp.where(m, b, a)
                .at[lane_ids ^ d].get(mode="promise_in_bounds"))

    def compute_chunk(c):
        bc = c % NBUF

        @plsc.parallel_loop(0, CHS[c] // 8,
                            carry=jnp.zeros((LANES,), jnp.float32))
        def half_body(h, carry):
            r0 = h * 8

            def dot_acc(j):
                r = r0 + j
                acc = (u_v[bc, r, pl.ds(0, LANES)]
                       * m_v[bc, r, pl.ds(0, LANES)])
                for k in range(1, D // LANES):
                    acc = acc + (u_v[bc, r, pl.ds(k * LANES, LANES)]
                                 * m_v[bc, r, pl.ds(k * LANES, LANES)])
                return acc

            w = comb(comb(comb(dot_acc(0), dot_acc(4), 4),
                          comb(dot_acc(2), dot_acc(6), 4), 2),
                     comb(comb(dot_acc(1), dot_acc(5), 4),
                          comb(dot_acc(3), dot_acc(7), 4), 2), 1)

            @pl.when(h & 1 == 1)
            def _():
                out_v[pl.ds(OFFS[c] + r0 - 8, LANES)] = comb(carry, w, 8)

            return w

    descs = [None] * NCHUNK
    s0.wait()
    s1.wait()
    descs[0] = start_gather(0)
    s2.wait()
    s3.wait()
    for c in range(1, min(NBUF, NCHUNK)):
        descs[c] = start_gather(c)
    outs = []
    for c in range(NCHUNK):
        for dsc in descs[c]:
            dsc.wait()
        compute_chunk(c)
        if c + NBUF < NCHUNK:
            descs[c + NBUF] = start_gather(c + NBUF)
        outs.append(pltpu.async_copy(
            out_v.at[pl.ds(OFFS[c], CHS[c])],
            out_hbm.at[pl.ds(base + OFFS[c], CHS[c])], sem6))
    for oc in outs:
        oc.wait()


def kernel(data, user_factors, movie_factors):
    users = data[:, 0].astype(jnp.int32)
    movies = data[:, 1].astype(jnp.int32)
    return _mf_kernel(users, movies, user_factors, movie_factors)


# speedup vs baseline: 1.0252x; 1.0017x over previous
import functools

import jax
import jax.numpy as jnp
from jax import lax
from jax.experimental import pallas as pl
from jax.experimental.pallas import tpu as pltpu
from jax.experimental.pallas import tpu_sc as plsc

B = 16384
D = 128
NC = 2
NS = 16
NW = NC * NS
BPW = B // NW
CHS = (16, 48, 64, 128, 128, 128)
OFFS = (0, 16, 64, 128, 256, 384)
NCHUNK = len(CHS)
CHMAX = max(CHS)
NBUF = 3
LANES = 16

_mesh = plsc.VectorSubcoreMesh(core_axis_name="c", subcore_axis_name="s")


@functools.partial(
    pl.kernel,
    mesh=_mesh,
    out_type=jax.ShapeDtypeStruct((B,), jnp.float32),
    scratch_types=[
        pltpu.VMEM((BPW,), jnp.int32),
        pltpu.VMEM((BPW,), jnp.int32),
        pltpu.VMEM((NBUF, CHMAX, D), jnp.float32),
        pltpu.VMEM((NBUF, CHMAX, D), jnp.float32),
        pltpu.VMEM((BPW,), jnp.float32),
        pltpu.SemaphoreType.DMA,
        pltpu.SemaphoreType.DMA,
        pltpu.SemaphoreType.DMA,
        pltpu.SemaphoreType.DMA,
        pltpu.SemaphoreType.DMA,
        pltpu.SemaphoreType.DMA,
        pltpu.SemaphoreType.DMA,
    ],
)
def _mf_kernel(users_hbm, movies_hbm, uf_hbm, mf_hbm, out_hbm,
               uidx_v, midx_v, u_v, m_v, out_v,
               sem0, sem1, sem2, sem3, sem4, sem5, sem6):
    wid = lax.axis_index("s") * NC + lax.axis_index("c")
    base = wid * BPW
    C0 = CHS[0]
    s0 = pltpu.async_copy(users_hbm.at[pl.ds(base, C0)],
                          uidx_v.at[pl.ds(0, C0)], sem0)
    s1 = pltpu.async_copy(movies_hbm.at[pl.ds(base, C0)],
                          midx_v.at[pl.ds(0, C0)], sem0)
    REST = BPW - C0
    s2 = pltpu.async_copy(users_hbm.at[pl.ds(base + C0, REST)],
                          uidx_v.at[pl.ds(C0, REST)], sem1)
    s3 = pltpu.async_copy(movies_hbm.at[pl.ds(base + C0, REST)],
                          midx_v.at[pl.ds(C0, REST)], sem1)

    lane_ids = lax.iota(jnp.int32, LANES)

    sems = (sem2, sem3, sem4)

    def start_gather(c):
        bc = c % NBUF
        n = CHS[c]
        nsplit = 2 if n >= 96 else 1
        h = n // nsplit
        copies = []
        for i in range(nsplit):
            copies.append(pltpu.async_copy(
                uf_hbm.at[uidx_v.at[pl.ds(OFFS[c] + i * h, h)]],
                u_v.at[bc, pl.ds(i * h, h)], sems[bc]))
            copies.append(pltpu.async_copy(
                mf_hbm.at[midx_v.at[pl.ds(OFFS[c] + i * h, h)]],
                m_v.at[bc, pl.ds(i * h, h)], sems[bc]))
        return tuple(copies)

    masks = {d: (lane_ids & d) == 0 for d in (8, 4, 2, 1)}

    def comb(a, b, d):
        m = masks[d]
        return (jnp.where(m, a, b)
                + jnp.where(m, b, a)
                .at[lane_ids ^ d].get(mode="promise_in_bounds"))

    def compute_chunk(c):
        bc = c % NBUF

        @plsc.parallel_loop(0, CHS[c] // 8, unroll=2,
                            carry=jnp.zeros((LANES,), jnp.float32))
        def half_body(h, carry):
            r0 = h * 8

            def dot_acc(j):
                r = r0 + j
                acc = (u_v[bc, r, pl.ds(0, LANES)]
                       * m_v[bc, r, pl.ds(0, LANES)])
                for k in range(1, D // LANES):
                    acc = acc + (u_v[bc, r, pl.ds(k * LANES, LANES)]
                                 * m_v[bc, r, pl.ds(k * LANES, LANES)])
                return acc

            w = comb(comb(comb(dot_acc(0), dot_acc(4), 4),
                          comb(dot_acc(2), dot_acc(6), 4), 2),
                     comb(comb(dot_acc(1), dot_acc(5), 4),
                          comb(dot_acc(3), dot_acc(7), 4), 2), 1)

            @pl.when(h & 1 == 1)
            def _():
                out_v[pl.ds(OFFS[c] + r0 - 8, LANES)] = comb(carry, w, 8)

            return w

    descs = [None] * NCHUNK
    s0.wait()
    s1.wait()
    descs[0] = start_gather(0)
    s2.wait()
    s3.wait()
    for c in range(1, min(NBUF, NCHUNK)):
        descs[c] = start_gather(c)
    outs = []
    for c in range(NCHUNK):
        for dsc in descs[c]:
            dsc.wait()
        compute_chunk(c)
        if c + NBUF < NCHUNK:
            descs[c + NBUF] = start_gather(c + NBUF)
        outs.append(pltpu.async_copy(
            out_v.at[pl.ds(OFFS[c], CHS[c])],
            out_hbm.at[pl.ds(base + OFFS[c], CHS[c])], sem6))
    for oc in outs:
        oc.wait()


def kernel(data, user_factors, movie_factors):
    users = data[:, 0].astype(jnp.int32)
    movies = data[:, 1].astype(jnp.int32)
    return _mf_kernel(users, movies, user_factors, movie_factors)


# speedup vs baseline: 1.0278x; 1.0026x over previous
import functools

import jax
import jax.numpy as jnp
from jax import lax
from jax.experimental import pallas as pl
from jax.experimental.pallas import tpu as pltpu
from jax.experimental.pallas import tpu_sc as plsc

B = 16384
D = 128
NC = 2
NS = 16
NW = NC * NS
BPW = B // NW
CHS = (16, 48, 64, 128, 128, 128)
OFFS = (0, 16, 64, 128, 256, 384)
NCHUNK = len(CHS)
CHMAX = max(CHS)
NBUF = 3
LANES = 16

_mesh = plsc.VectorSubcoreMesh(core_axis_name="c", subcore_axis_name="s")


@functools.partial(
    pl.kernel,
    mesh=_mesh,
    out_type=jax.ShapeDtypeStruct((B,), jnp.float32),
    scratch_types=[
        pltpu.VMEM((BPW,), jnp.int32),
        pltpu.VMEM((BPW,), jnp.int32),
        pltpu.VMEM((NBUF, CHMAX, D), jnp.float32),
        pltpu.VMEM((NBUF, CHMAX, D), jnp.float32),
        pltpu.VMEM((BPW,), jnp.float32),
        pltpu.SemaphoreType.DMA,
        pltpu.SemaphoreType.DMA,
        pltpu.SemaphoreType.DMA,
        pltpu.SemaphoreType.DMA,
        pltpu.SemaphoreType.DMA,
        pltpu.SemaphoreType.DMA,
        pltpu.SemaphoreType.DMA,
        pltpu.SemaphoreType.DMA,
        pltpu.SemaphoreType.DMA,
    ],
)
def _mf_kernel(users_hbm, movies_hbm, uf_hbm, mf_hbm, out_hbm,
               uidx_v, midx_v, u_v, m_v, out_v,
               sem_s0, sem_s1, semc0, semc1, semc2, semc3, semc4, semc5,
               sem_out):
    wid = lax.axis_index("s") * NC + lax.axis_index("c")
    base = wid * BPW
    C0 = CHS[0]
    s0 = pltpu.async_copy(users_hbm.at[pl.ds(base, C0)],
                          uidx_v.at[pl.ds(0, C0)], sem_s0)
    s1 = pltpu.async_copy(movies_hbm.at[pl.ds(base, C0)],
                          midx_v.at[pl.ds(0, C0)], sem_s0)
    REST = BPW - C0
    s2 = pltpu.async_copy(users_hbm.at[pl.ds(base + C0, REST)],
                          uidx_v.at[pl.ds(C0, REST)], sem_s1)
    s3 = pltpu.async_copy(movies_hbm.at[pl.ds(base + C0, REST)],
                          midx_v.at[pl.ds(C0, REST)], sem_s1)

    lane_ids = lax.iota(jnp.int32, LANES)

    sems = (semc0, semc1, semc2, semc3, semc4, semc5)

    def start_gather(c):
        bc = c % NBUF
        n = CHS[c]
        nsplit = 2 if n >= 96 else 1
        h = n // nsplit
        copies = []
        for i in range(nsplit):
            copies.append(pltpu.async_copy(
                uf_hbm.at[uidx_v.at[pl.ds(OFFS[c] + i * h, h)]],
                u_v.at[bc, pl.ds(i * h, h)], sems[c]))
            copies.append(pltpu.async_copy(
                mf_hbm.at[midx_v.at[pl.ds(OFFS[c] + i * h, h)]],
                m_v.at[bc, pl.ds(i * h, h)], sems[c]))
        return tuple(copies)

    masks = {d: (lane_ids & d) == 0 for d in (8, 4, 2, 1)}

    def comb(a, b, d):
        m = masks[d]
        return (jnp.where(m, a, b)
                + jnp.where(m, b, a)
                .at[lane_ids ^ d].get(mode="promise_in_bounds"))

    def compute_chunk(c):
        bc = c % NBUF

        def half_body(h, carry):
            r0 = h * 8

            def dot_acc(j):
                r = r0 + j
                acc = (u_v[bc, r, pl.ds(0, LANES)]
                       * m_v[bc, r, pl.ds(0, LANES)])
                for k in range(1, D // LANES):
                    acc = acc + (u_v[bc, r, pl.ds(k * LANES, LANES)]
                                 * m_v[bc, r, pl.ds(k * LANES, LANES)])
                return acc

            w = comb(comb(comb(dot_acc(0), dot_acc(4), 4),
                          comb(dot_acc(2), dot_acc(6), 4), 2),
                     comb(comb(dot_acc(1), dot_acc(5), 4),
                          comb(dot_acc(3), dot_acc(7), 4), 2), 1)

            @pl.when(h & 1 == 1)
            def _():
                out_v[pl.ds(OFFS[c] + r0 - 8, LANES)] = comb(carry, w, 8)

            return w

        lax.fori_loop(0, CHS[c] // 8, half_body,
                      jnp.zeros((LANES,), jnp.float32))

    descs = [None] * NCHUNK
    s0.wait()
    s1.wait()
    descs[0] = start_gather(0)
    s2.wait()
    s3.wait()
    for c in range(1, min(NBUF, NCHUNK)):
        descs[c] = start_gather(c)
    for c in range(NCHUNK):
        for dsc in descs[c]:
            dsc.wait()
        compute_chunk(c)
        if c + NBUF < NCHUNK:
            descs[c + NBUF] = start_gather(c + NBUF)

    pltpu.sync_copy(out_v, out_hbm.at[pl.ds(base, BPW)])


def kernel(data, user_factors, movie_factors):
    users = data[:, 0].astype(jnp.int32)
    movies = data[:, 1].astype(jnp.int32)
    return _mf_kernel(users, movies, user_factors, movie_factors)


# speedup vs baseline: 1.0316x; 1.0037x over previous
import functools

import jax
import jax.numpy as jnp
from jax import lax
from jax.experimental import pallas as pl
from jax.experimental.pallas import tpu as pltpu
from jax.experimental.pallas import tpu_sc as plsc

B = 16384
D = 128
NC = 2
NS = 16
NW = NC * NS
BPW = B // NW
CHS = (16, 48, 64, 128, 128, 128)
OFFS = (0, 16, 64, 128, 256, 384)
NCHUNK = len(CHS)
CHMAX = max(CHS)
NBUF = 3
LANES = 16

_mesh = plsc.VectorSubcoreMesh(core_axis_name="c", subcore_axis_name="s")


@functools.partial(
    pl.kernel,
    mesh=_mesh,
    out_type=jax.ShapeDtypeStruct((B,), jnp.float32),
    scratch_types=[
        pltpu.VMEM((BPW,), jnp.int32),
        pltpu.VMEM((BPW,), jnp.int32),
        pltpu.VMEM((NBUF, CHMAX, D), jnp.float32),
        pltpu.VMEM((NBUF, CHMAX, D), jnp.float32),
        pltpu.VMEM((BPW,), jnp.float32),
        pltpu.SemaphoreType.DMA,
        pltpu.SemaphoreType.DMA,
        pltpu.SemaphoreType.DMA,
        pltpu.SemaphoreType.DMA,
        pltpu.SemaphoreType.DMA,
        pltpu.SemaphoreType.DMA,
        pltpu.SemaphoreType.DMA,
        pltpu.SemaphoreType.DMA,
        pltpu.SemaphoreType.DMA,
    ],
)
def _mf_kernel(users_hbm, movies_hbm, uf_hbm, mf_hbm, out_hbm,
               uidx_v, midx_v, u_v, m_v, out_v,
               sem_s0, sem_s1, semc0, semc1, semc2, semc3, semc4, semc5,
               sem_out):
    wid = lax.axis_index("s") * NC + lax.axis_index("c")
    base = wid * BPW
    C0 = CHS[0]
    s0 = pltpu.async_copy(users_hbm.at[pl.ds(base, C0)],
                          uidx_v.at[pl.ds(0, C0)], sem_s0)
    s1 = pltpu.async_copy(movies_hbm.at[pl.ds(base, C0)],
                          midx_v.at[pl.ds(0, C0)], sem_s0)
    REST = BPW - C0
    s2 = pltpu.async_copy(users_hbm.at[pl.ds(base + C0, REST)],
                          uidx_v.at[pl.ds(C0, REST)], sem_s1)
    s3 = pltpu.async_copy(movies_hbm.at[pl.ds(base + C0, REST)],
                          midx_v.at[pl.ds(C0, REST)], sem_s1)

    lane_ids = lax.iota(jnp.int32, LANES)

    sems = (semc0, semc1, semc2, semc3, semc4, semc5)

    def start_gather(c):
        bc = c % NBUF
        n = CHS[c]
        nsplit = 2 if n >= 96 else 1
        h = n // nsplit
        copies = []
        for i in range(nsplit):
            copies.append(pltpu.async_copy(
                uf_hbm.at[uidx_v.at[pl.ds(OFFS[c] + i * h, h)]],
                u_v.at[bc, pl.ds(i * h, h)], sems[c]))
            copies.append(pltpu.async_copy(
                mf_hbm.at[midx_v.at[pl.ds(OFFS[c] + i * h, h)]],
                m_v.at[bc, pl.ds(i * h, h)], sems[c]))
        return tuple(copies)

    masks = {d: (lane_ids & d) == 0 for d in (8, 4, 2, 1)}

    def comb(a, b, d):
        m = masks[d]
        return (jnp.where(m, a, b)
                + jnp.where(m, b, a)
                .at[lane_ids ^ d].get(mode="promise_in_bounds"))

    def compute_chunk(c):
        bc = c % NBUF

        def half_body(h, carry):
            r0 = h * 8

            def dot_acc(j):
                r = r0 + j
                acc = (u_v[bc, r, pl.ds(0, LANES)]
                       * m_v[bc, r, pl.ds(0, LANES)])
                for k in range(1, D // LANES):
                    acc = acc + (u_v[bc, r, pl.ds(k * LANES, LANES)]
                                 * m_v[bc, r, pl.ds(k * LANES, LANES)])
                return acc

            w = comb(comb(comb(dot_acc(0), dot_acc(4), 4),
                          comb(dot_acc(2), dot_acc(6), 4), 2),
                     comb(comb(dot_acc(1), dot_acc(5), 4),
                          comb(dot_acc(3), dot_acc(7), 4), 2), 1)

            @pl.when(h & 1 == 1)
            def _():
                out_v[pl.ds(OFFS[c] + r0 - 8, LANES)] = comb(carry, w, 8)

            return w

        lax.fori_loop(0, CHS[c] // 8, half_body,
                      jnp.zeros((LANES,), jnp.float32))

    descs = [None] * NCHUNK
    s0.wait()
    s1.wait()
    descs[0] = start_gather(0)
    s2.wait()
    s3.wait()
    for c in range(1, min(NBUF, NCHUNK)):
        descs[c] = start_gather(c)
    for c in range(NCHUNK):
        for dsc in descs[c]:
            dsc.wait()
        compute_chunk(c)
        if c + NBUF < NCHUNK:
            descs[c + NBUF] = start_gather(c + NBUF)

    plsc.subcore_barrier()
    pltpu.sync_copy(out_v, out_hbm.at[pl.ds(base, BPW)])


def kernel(data, user_factors, movie_factors):
    users = data[:, 0].astype(jnp.int32)
    movies = data[:, 1].astype(jnp.int32)
    return _mf_kernel(users, movies, user_factors, movie_factors)
